# baseline (speedup 1.0000x reference)
import functools

import jax
import jax.numpy as jnp
from jax import lax
from jax.experimental import pallas as pl
from jax.experimental.pallas import tpu as pltpu
from jax.experimental.pallas import tpu_sc as plsc

B = 4096
S = 200
D = 64
H = 256
C = 50
CPAD = 128

NC = 2
NS = 16
NW = NC * NS
BPW = B // NW
SP = 256
LANES = 16
DCH = D // LANES
NBUF = 4
SGRP = 16
NG = 13
SPAD = NG * SGRP


def _sc_bag(X, table):
  mesh = plsc.VectorSubcoreMesh(core_axis_name="c", subcore_axis_name="s")

  @functools.partial(
      pl.kernel,
      out_type=jax.ShapeDtypeStruct((B, D), jnp.float32),
      mesh=mesh,
      compiler_params=pltpu.CompilerParams(use_tc_tiling_on_sc=True),
      scratch_types=[
          pltpu.VMEM((NBUF, SP), jnp.int32),
          pltpu.VMEM((NBUF * SPAD, D), jnp.float32),
          pltpu.VMEM((BPW, D), jnp.float32),
          pltpu.SemaphoreType.DMA((NBUF,)),
          pltpu.SemaphoreType.DMA((NBUF,)),
      ],
  )
  def bag(x_hbm, tab_hbm, out_hbm, idx_v, rows_v, out_v, sems, isems):
    w = lax.axis_index("s") * NC + lax.axis_index("c")
    base = w * BPW

    def fetch_idx(b, slot):
      pltpu.async_copy(x_hbm.at[base + b], idx_v.at[slot], isems.at[slot])

    def issue(b, slot):
      pltpu.make_async_copy(
          x_hbm.at[0], idx_v.at[slot], isems.at[slot]
      ).wait()

      @pl.loop(0, NG)
      def _(jg):
        vec = idx_v[slot, pl.ds(jg * SGRP, SGRP)]
        for k in range(SGRP):
          pltpu.async_copy(
              tab_hbm.at[vec[k]],
              rows_v.at[slot * SPAD + jg * SGRP + k],
              sems.at[slot],
          )

      @pl.when(b + NBUF < BPW)
      def _():
        fetch_idx(b + NBUF, slot)

    def wait_rows(slot):
      pltpu.make_async_copy(
          tab_hbm.at[pl.ds(0, SPAD)],
          rows_v.at[pl.ds(slot * SPAD, SPAD)],
          sems.at[slot],
      ).wait()

    def accum(slot):
      def body(j, accs):
        return tuple(
            accs[c] + rows_v[slot * SPAD + j, pl.ds(c * LANES, LANES)]
            for c in range(DCH)
        )
      zeros = tuple(jnp.zeros((LANES,), jnp.float32) for _ in range(DCH))
      return lax.fori_loop(0, S, body, zeros)

    for slot in range(NBUF):
      fetch_idx(slot, slot)
    for slot in range(NBUF):
      issue(slot, slot)

    @pl.loop(0, BPW, step=NBUF)
    def _(b):
      for slot in range(NBUF):
        wait_rows(slot)
        acc = accum(slot)

        @pl.when(b + slot + NBUF < BPW)
        def _():
          issue(b + slot + NBUF, slot)

        for c in range(DCH):
          out_v[b + slot, pl.ds(c * LANES, LANES)] = acc[c] * (1.0 / S)

    pltpu.sync_copy(out_v, out_hbm.at[pl.ds(base, BPW)])

  return bag(X, table)


def _mlp_body(x_ref, w1_ref, b1_ref, w2_ref, b2_ref, o_ref):
  x = x_ref[...]
  h = jnp.maximum(
      jnp.dot(x, w1_ref[...], preferred_element_type=jnp.float32) + b1_ref[...],
      0.0,
  )
  logits = (
      jnp.dot(h, w2_ref[...], preferred_element_type=jnp.float32) + b2_ref[...]
  )
  m = jnp.max(logits, axis=-1, keepdims=True)
  s = logits - m
  lse = jnp.log(jnp.sum(jnp.exp(s), axis=-1, keepdims=True))
  o_ref[...] = s - lse


def _mlp(bag, W1, b1, W2p, b2p):
  BB = 512
  return pl.pallas_call(
      _mlp_body,
      grid=(B // BB,),
      in_specs=[
          pl.BlockSpec((BB, D), lambda i: (i, 0)),
          pl.BlockSpec((D, H), lambda i: (0, 0)),
          pl.BlockSpec((1, H), lambda i: (0, 0)),
          pl.BlockSpec((H, CPAD), lambda i: (0, 0)),
          pl.BlockSpec((1, CPAD), lambda i: (0, 0)),
      ],
      out_specs=pl.BlockSpec((BB, CPAD), lambda i: (i, 0)),
      out_shape=jax.ShapeDtypeStruct((B, CPAD), jnp.float32),
  )(bag, W1, b1, W2p, b2p)


@jax.jit
def kernel(X, table, W1, b1, W2, b2):
  Xp = jnp.pad(X, ((0, 0), (0, SP - S)))
  bag = _sc_bag(Xp, table)
  W2p = jnp.pad(W2, ((0, 0), (0, CPAD - C)))
  b2p = jnp.pad(b2, (0, CPAD - C), constant_values=-1e30).reshape(1, CPAD)
  out = _mlp(bag, W1, b1.reshape(1, H), W2p, b2p)
  return out[:, :C]


# speedup vs baseline: 2.2124x; 2.2124x over previous
import functools

import jax
import jax.numpy as jnp
from jax import lax
from jax.experimental import pallas as pl
from jax.experimental.pallas import tpu as pltpu
from jax.experimental.pallas import tpu_sc as plsc

B = 4096
S = 200
D = 64
V = 1000000
H = 256
C = 50
CPAD = 128

NC = 2
NS = 16
NW = NC * NS
BPW = B // NW
SA = 128
SB = S - SA
SP = 256
DP = 128
LANES = 16
DCH = D // LANES
NT = 2048


def _linearize_body(tt_ref, o_ref):
  t = jnp.swapaxes(tt_ref[...], 0, 1)
  o_ref[...] = jnp.concatenate([t, t], axis=1)


def _linearize(tableT):
  return pl.pallas_call(
      _linearize_body,
      grid=((V + NT - 1) // NT,),
      in_specs=[pl.BlockSpec((D, NT), lambda i: (0, i))],
      out_specs=pl.BlockSpec((NT, DP), lambda i: (i, 0)),
      out_shape=jax.ShapeDtypeStruct((V, DP), jnp.float32),
  )(tableT)


def _sc_bag(X, table):
  mesh = plsc.VectorSubcoreMesh(core_axis_name="c", subcore_axis_name="s")

  @functools.partial(
      pl.kernel,
      out_type=jax.ShapeDtypeStruct((B, D), jnp.float32),
      mesh=mesh,
      compiler_params=pltpu.CompilerParams(use_tc_tiling_on_sc=True),
      scratch_types=[
          pltpu.VMEM((BPW, SP), jnp.int32),
          pltpu.VMEM((SA, DP), jnp.float32),
          pltpu.VMEM((SB, DP), jnp.float32),
          pltpu.VMEM((BPW, D), jnp.float32),
          pltpu.SemaphoreType.DMA,
          pltpu.SemaphoreType.DMA,
      ],
  )
  def bag(x_hbm, tab_hbm, out_hbm, idx_v, buf_a, buf_b, out_v, sem_a, sem_b):
    w = lax.axis_index("s") * NC + lax.axis_index("c")
    base = w * BPW
    pltpu.sync_copy(x_hbm.at[pl.ds(base, BPW)], idx_v)

    def start_a(b):
      pltpu.async_copy(tab_hbm.at[idx_v.at[b, pl.ds(0, SA)]], buf_a, sem_a)

    def start_b(b):
      pltpu.async_copy(tab_hbm.at[idx_v.at[b, pl.ds(SA, SB)]], buf_b, sem_b)

    def wait(idx_slice, buf, sem):
      pltpu.make_async_copy(tab_hbm.at[idx_slice], buf, sem).wait()

    def accum(buf, n, accs):
      def body(r, accs):
        return tuple(
            accs[c] + buf[r, pl.ds(c * LANES, LANES)] for c in range(DCH)
        )
      return lax.fori_loop(0, n, body, accs)

    start_a(0)
    start_b(0)

    @pl.loop(0, BPW)
    def _(b):
      zeros = tuple(jnp.zeros((LANES,), jnp.float32) for _ in range(DCH))
      wait(idx_v.at[0, pl.ds(0, SA)], buf_a, sem_a)
      acc = accum(buf_a, SA, zeros)

      @pl.when(b < BPW - 1)
      def _():
        start_a(b + 1)

      wait(idx_v.at[0, pl.ds(SA, SB)], buf_b, sem_b)
      acc = accum(buf_b, SB, acc)

      @pl.when(b < BPW - 1)
      def _():
        start_b(b + 1)

      for c in range(DCH):
        out_v[b, pl.ds(c * LANES, LANES)] = acc[c] * (1.0 / S)

    pltpu.sync_copy(out_v, out_hbm.at[pl.ds(base, BPW)])

  return bag(X, table)


def _mlp_body(x_ref, w1_ref, b1_ref, w2_ref, b2_ref, o_ref):
  x = x_ref[...]
  h = jnp.maximum(
      jnp.dot(x, w1_ref[...], preferred_element_type=jnp.float32) + b1_ref[...],
      0.0,
  )
  logits = (
      jnp.dot(h, w2_ref[...], preferred_element_type=jnp.float32) + b2_ref[...]
  )
  m = jnp.max(logits, axis=-1, keepdims=True)
  s = logits - m
  lse = jnp.log(jnp.sum(jnp.exp(s), axis=-1, keepdims=True))
  o_ref[...] = s - lse


def _mlp(bag, W1, b1, W2p, b2p):
  BB = 512
  return pl.pallas_call(
      _mlp_body,
      grid=(B // BB,),
      in_specs=[
          pl.BlockSpec((BB, D), lambda i: (i, 0)),
          pl.BlockSpec((D, H), lambda i: (0, 0)),
          pl.BlockSpec((1, H), lambda i: (0, 0)),
          pl.BlockSpec((H, CPAD), lambda i: (0, 0)),
          pl.BlockSpec((1, CPAD), lambda i: (0, 0)),
      ],
      out_specs=pl.BlockSpec((BB, CPAD), lambda i: (i, 0)),
      out_shape=jax.ShapeDtypeStruct((B, CPAD), jnp.float32),
  )(bag, W1, b1, W2p, b2p)


@jax.jit
def kernel(X, table, W1, b1, W2, b2):
  Xp = jnp.pad(X, ((0, 0), (0, SP - S)))
  tp = _linearize(table.T)
  bag = _sc_bag(Xp, tp)
  W2p = jnp.pad(W2, ((0, 0), (0, CPAD - C)))
  b2p = jnp.pad(b2, (0, CPAD - C), constant_values=-1e30).reshape(1, CPAD)
  out = _mlp(bag, W1, b1.reshape(1, H), W2p, b2p)
  return out[:, :C]
